# Initial kernel scaffold; baseline (speedup 1.0000x reference)
import jax
import jax.numpy as jnp
from jax.experimental import pallas as pl


def kernel(atom_features, edge_index, batch, W1l, b1, W1r, gn1, bn1, W2l, b2, W2r, Wout, bout, g_ln, b_ln):
    raise NotImplementedError("write your pallas kernel here")


# speedup vs baseline: 1.0002x; 1.0002x over previous
import jax
import jax.numpy as jnp
from jax.experimental import pallas as pl

N_NODES = 10000
N_GRAPHS = 256
D = 128


def _head_kernel(p_ref, w_ref, b_ref, g_ref, bb_ref, o_ref):
    y = jnp.dot(p_ref[...], w_ref[...].T, preferred_element_type=jnp.float32)
    y = y + b_ref[...]
    y = jnp.maximum(y, 0.0)
    mu = jnp.mean(y, axis=-1, keepdims=True)
    var = jnp.mean((y - mu) ** 2, axis=-1, keepdims=True)
    o_ref[...] = (y - mu) / jnp.sqrt(var + 1e-5) * g_ref[...] + bb_ref[...]


def _sage(x, src, dst, Wl, bl, Wr):
    msg = jnp.take(x, src, axis=0)
    s = jax.ops.segment_sum(msg, dst, num_segments=N_NODES)
    cnt = jax.ops.segment_sum(jnp.ones((src.shape[0],), x.dtype), dst, num_segments=N_NODES)
    mean = s / jnp.maximum(cnt, 1.0)[:, None]
    return mean @ Wl.T + bl + x @ Wr.T


def kernel(atom_features, edge_index, batch, W1l, b1, W1r, gn1, bn1, W2l, b2, W2r, Wout, bout, g_ln, b_ln):
    src = edge_index[0]
    dst = edge_index[1]
    x = _sage(atom_features, src, dst, W1l, b1, W1r)
    mu = jnp.mean(x, axis=-1, keepdims=True)
    var = jnp.var(x, axis=-1, keepdims=True)
    x = (x - mu) / jnp.sqrt(var + 1e-5) * gn1 + bn1
    x = jax.nn.relu(x)
    x = _sage(x, src, dst, W2l, b2, W2r)
    pooled = jax.ops.segment_max(x, batch, num_segments=N_GRAPHS)
    return pl.pallas_call(
        _head_kernel,
        out_shape=jax.ShapeDtypeStruct((N_GRAPHS, D), jnp.float32),
    )(pooled, Wout, bout.reshape(1, D), g_ln.reshape(1, D), b_ln.reshape(1, D))


# speedup vs baseline: 8.0252x; 8.0232x over previous
import functools

import jax
import jax.numpy as jnp
from jax import lax
from jax.experimental import pallas as pl
from jax.experimental.pallas import tpu as pltpu
from jax.experimental.pallas import tpu_sc as plsc

N = 10000
E = 320000
G = 256
D = 128
NC = 2
NS = 16
NW = NC * NS
EPW = E // NW
CHUNK = 125
NCH = EPW // CHUNK
STG = 16
NSTG = NCH // STG
N_PAD = 10240
RPW = N_PAD // NS
CNT_PAD = 10240
CPW = CNT_PAD // NS
GPW = G // NW
RB = 64
RB1 = 1000

_mesh = plsc.VectorSubcoreMesh(core_axis_name="c", subcore_axis_name="s")


def _agg_body(with_cnt, *refs):
    if with_cnt:
        (x_hbm, src_hbm, dst_hbm, zrow_hbm, zcnt_hbm, ones_hbm,
         s_out, cnt_out, s_sh, cnt_sh, stg_src, stg_dst, gbuf, ones_v, gsem) = refs
    else:
        (x_hbm, src_hbm, dst_hbm, zrow_hbm,
         s_out, s_sh, stg_src, stg_dst, gbuf, gsem) = refs
    cid = lax.axis_index("c")
    sid = lax.axis_index("s")
    wid = cid * NS + sid

    pltpu.sync_copy(zrow_hbm, s_sh.at[pl.ds(sid * RPW, RPW)])
    if with_cnt:
        pltpu.sync_copy(zcnt_hbm, cnt_sh.at[pl.ds(sid * CPW, CPW)])
        pltpu.sync_copy(ones_hbm, ones_v)
    plsc.subcore_barrier()

    def stage(s, _):
        cb = wid * NCH + s * STG
        pltpu.sync_copy(src_hbm.at[pl.ds(cb, STG)], stg_src)
        pltpu.sync_copy(dst_hbm.at[pl.ds(cb, STG)], stg_dst)

        def chunk(k, _):
            pltpu.async_copy(x_hbm.at[stg_src.at[k]], gbuf, gsem).wait()
            pltpu.sync_copy(gbuf, s_sh.at[stg_dst.at[k]], add=True)
            if with_cnt:
                pltpu.sync_copy(ones_v, cnt_sh.at[stg_dst.at[k]], add=True)
            return 0

        lax.fori_loop(0, STG, chunk, 0)
        return 0

    lax.fori_loop(0, NSTG, stage, 0)
    plsc.subcore_barrier()

    for t in range(5):
        r = sid * RPW + t * 128
        pltpu.sync_copy(s_sh.at[pl.ds(r, 128)], s_out.at[pl.ds(cid * N_PAD + r, 128)])
    if with_cnt:
        pltpu.sync_copy(cnt_sh.at[pl.ds(sid * CPW, CPW)],
                        cnt_out.at[pl.ds(cid * CNT_PAD + sid * CPW, CPW)])


def _make_agg(with_cnt):
    out_type = [jax.ShapeDtypeStruct((NC * N_PAD, D), jnp.float32)]
    scratch = [
        pltpu.VMEM_SHARED((N_PAD, D), jnp.float32),
        pltpu.VMEM((STG, CHUNK), jnp.int32),
        pltpu.VMEM((STG, CHUNK), jnp.int32),
        pltpu.VMEM((CHUNK, D), jnp.float32),
        pltpu.SemaphoreType.DMA,
    ]
    if with_cnt:
        out_type.append(jax.ShapeDtypeStruct((NC * CNT_PAD,), jnp.float32))
        scratch = ([scratch[0], pltpu.VMEM_SHARED((CNT_PAD,), jnp.float32)]
                   + scratch[1:-1]
                   + [pltpu.VMEM((CHUNK,), jnp.float32), pltpu.SemaphoreType.DMA])
    return pl.kernel(
        functools.partial(_agg_body, with_cnt),
        out_type=out_type,
        mesh=_mesh,
        scratch_types=scratch,
    )


def _sload(ref, idx):
    return ref[pl.ds(idx, 16)][0]


def _pool_body(h_hbm, batch_hbm, ninf_hbm, out_hbm, batch_v, rbuf, acc, sem):
    cid = lax.axis_index("c")
    sid = lax.axis_index("s")
    wid = cid * NS + sid
    g0 = wid * GPW
    pltpu.sync_copy(batch_hbm, batch_v.at[pl.ds(0, N)])
    pltpu.sync_copy(ninf_hbm, acc)

    def lower_bound(tgt):
        pos = jnp.int32(0)
        w = 16384
        while w:
            cand = pos + w
            v = _sload(batch_v, jnp.minimum(cand, N) - 1)
            pos = jnp.where((cand <= N) & (v < tgt), cand, pos)
            w //= 2
        return pos

    lo = (lower_bound(g0) // 8) * 8
    hi = lower_bound(g0 + GPW)

    nwin = (hi - lo + RB - 1) // RB

    def step(k, _):
        rc = pl.multiple_of(jnp.minimum(lo + k * RB, N - RB), 8)
        pltpu.sync_copy(h_hbm.at[pl.ds(rc, RB)], rbuf)

        def row(i, _):
            g = _sload(batch_v, rc + i)
            gl = g - g0
            valid = (gl >= 0) & (gl < GPW)
            glc = jnp.clip(gl, 0, GPW - 1)
            for j in range(D // 16):
                cur = acc[glc, pl.ds(j * 16, 16)]
                new = jnp.maximum(cur, rbuf[i, pl.ds(j * 16, 16)])
                acc[glc, pl.ds(j * 16, 16)] = jnp.where(valid, new, cur)
            return 0

        lax.fori_loop(0, RB, row, 0)
        return 0

    lax.fori_loop(0, nwin, step, 0)
    pltpu.sync_copy(acc, out_hbm.at[pl.ds(g0, GPW)])


_pool = pl.kernel(
    _pool_body,
    out_type=jax.ShapeDtypeStruct((G, D), jnp.float32),
    mesh=_mesh,
    scratch_types=[
        pltpu.VMEM((N + 16,), jnp.int32),
        pltpu.VMEM((RB, D), jnp.float32),
        pltpu.VMEM((GPW, D), jnp.float32),
        pltpu.SemaphoreType.DMA,
    ],
)

_CONTRACT_T = (((1,), (1,)), ((), ()))


def _dense_block(apply_ln, s_ref, cnt_ref, x_ref, wl_ref, wr_ref, b_ref,
                 g_ref, bb_ref, o_ref):
    s = s_ref[0] + s_ref[1]
    cnt = cnt_ref[0] + cnt_ref[1]
    mean = s / jnp.maximum(cnt, 1.0)
    h = (lax.dot_general(mean, wl_ref[...], _CONTRACT_T,
                         preferred_element_type=jnp.float32)
         + lax.dot_general(x_ref[...], wr_ref[...], _CONTRACT_T,
                           preferred_element_type=jnp.float32)
         + b_ref[...])
    if apply_ln:
        mu = jnp.mean(h, axis=-1, keepdims=True)
        var = jnp.mean((h - mu) ** 2, axis=-1, keepdims=True)
        h = (h - mu) * lax.rsqrt(var + 1e-5) * g_ref[...] + bb_ref[...]
        h = jnp.maximum(h, 0.0)
    o_ref[...] = h


def _make_dense(apply_ln):
    return pl.pallas_call(
        functools.partial(_dense_block, apply_ln),
        grid=(N // RB1,),
        in_specs=[
            pl.BlockSpec((NC, RB1, D), lambda i: (0, i, 0)),
            pl.BlockSpec((NC, RB1, 1), lambda i: (0, i, 0)),
            pl.BlockSpec((RB1, D), lambda i: (i, 0)),
            pl.BlockSpec((D, D), lambda i: (0, 0)),
            pl.BlockSpec((D, D), lambda i: (0, 0)),
            pl.BlockSpec((1, D), lambda i: (0, 0)),
            pl.BlockSpec((1, D), lambda i: (0, 0)),
            pl.BlockSpec((1, D), lambda i: (0, 0)),
        ],
        out_specs=pl.BlockSpec((RB1, D), lambda i: (i, 0)),
        out_shape=jax.ShapeDtypeStruct((N, D), jnp.float32),
    )


def _head_block(p_ref, w_ref, b_ref, g_ref, bb_ref, o_ref):
    y = lax.dot_general(p_ref[...], w_ref[...], _CONTRACT_T,
                        preferred_element_type=jnp.float32) + b_ref[...]
    y = jnp.maximum(y, 0.0)
    mu = jnp.mean(y, axis=-1, keepdims=True)
    var = jnp.mean((y - mu) ** 2, axis=-1, keepdims=True)
    o_ref[...] = (y - mu) * lax.rsqrt(var + 1e-5) * g_ref[...] + bb_ref[...]


_head = pl.pallas_call(
    _head_block,
    out_shape=jax.ShapeDtypeStruct((G, D), jnp.float32),
)

_agg1 = _make_agg(True)
_agg2 = _make_agg(False)
_dense1 = _make_dense(True)
_dense2 = _make_dense(False)


def kernel(atom_features, edge_index, batch, W1l, b1, W1r, gn1, bn1,
           W2l, b2, W2r, Wout, bout, g_ln, b_ln):
    src2 = edge_index[0].reshape(E // CHUNK, CHUNK)
    dst2 = edge_index[1].reshape(E // CHUNK, CHUNK)
    zrow = jnp.zeros((RPW, D), jnp.float32)
    zcnt = jnp.zeros((CPW,), jnp.float32)
    ones = jnp.ones((CHUNK,), jnp.float32)
    ninf = jnp.full((GPW, D), float("-inf"), jnp.float32)

    s1, cnt = _agg1(atom_features, src2, dst2, zrow, zcnt, ones)
    s1 = s1.reshape(NC, N_PAD, D)
    cnt3 = cnt.reshape(NC, CNT_PAD, 1)
    b1r = b1.reshape(1, D)
    h1 = _dense1(s1, cnt3, atom_features, W1l, W1r, b1r,
                 gn1.reshape(1, D), bn1.reshape(1, D))
    (s2,) = _agg2(h1, src2, dst2, zrow)
    s2 = s2.reshape(NC, N_PAD, D)
    h2 = _dense2(s2, cnt3, h1, W2l, W2r, b2.reshape(1, D),
                 gn1.reshape(1, D), bn1.reshape(1, D))
    pooled = _pool(h2, batch, ninf)
    return _head(pooled, Wout, bout.reshape(1, D),
                 g_ln.reshape(1, D), b_ln.reshape(1, D))
